# Initial kernel scaffold; baseline (speedup 1.0000x reference)
import functools

import jax
import jax.numpy as jnp
from jax import lax
from jax.experimental import pallas as pl
from jax.experimental.pallas import tpu as pltpu
from jax.experimental.pallas import tpu_sc as plsc

N = 10000
NPAD = 10016
D_IN = 128
H1, C1 = 8, 16
H2, C2 = 1, 2
NEG = -1e30
CHUNK = 128
NTILES = 32
RPT = NPAD // 16

_HIGH = lax.Precision.HIGHEST


def _cdiv(a, b):
    return (a + b - 1) // b


def _prep_body(x_ref, w_ref, asrc_ref, adst_ref, h_ref, as_ref, ad_ref, mh_ref):
    x = x_ref[...]
    h = jnp.dot(x, w_ref[...], precision=_HIGH)
    h_ref[...] = h
    als = jnp.dot(h, asrc_ref[...], precision=_HIGH)
    ald = jnp.dot(h, adst_ref[...], precision=_HIGH)
    nh = als.shape[1]
    pad = jnp.full((als.shape[0], 16 - nh), NEG, jnp.float32)
    as_ref[...] = jnp.concatenate([als, pad], axis=1)
    ad_ref[...] = jnp.concatenate([ald, pad], axis=1)
    mh = jnp.max(als, axis=0) + jnp.max(ald, axis=0)
    mh_ref[...] = jnp.concatenate([mh, jnp.zeros((16 - nh,), jnp.float32)])[None, :]


def _tc_prep(xp, w, asrc, adst, hdim):
    return pl.pallas_call(
        _prep_body,
        out_shape=[
            jax.ShapeDtypeStruct((NPAD, hdim), jnp.float32),
            jax.ShapeDtypeStruct((NPAD, 16), jnp.float32),
            jax.ShapeDtypeStruct((NPAD, 16), jnp.float32),
            jax.ShapeDtypeStruct((1, 16), jnp.float32),
        ],
    )(xp, w, asrc, adst)


def _recip_body(den_ref, r_ref):
    r_ref[...] = 1.0 / (den_ref[0] + den_ref[1] + 1e-16)


def _tc_recip(den):
    return pl.pallas_call(
        _recip_body,
        out_shape=jax.ShapeDtypeStruct((NPAD, 16), jnp.float32),
    )(den)


def _mid_body(p_ref, b1_ref, w2_ref, asrc_ref, adst_ref,
              h2_ref, as_ref, ad_ref, mh_ref):
    o = p_ref[0] + p_ref[1] + b1_ref[...]
    o = jnp.where(o > 0, o, jnp.exp(jnp.minimum(o, 0.0)) - 1.0)
    h2 = jnp.dot(o, w2_ref[...], precision=_HIGH)
    h2_ref[...] = jnp.concatenate(
        [h2, jnp.zeros((h2.shape[0], 16 - h2.shape[1]), jnp.float32)], axis=1)
    als = jnp.dot(o, asrc_ref[...], precision=_HIGH)
    ald = jnp.dot(o, adst_ref[...], precision=_HIGH)
    pad = jnp.full((als.shape[0], 15), NEG, jnp.float32)
    as_ref[...] = jnp.concatenate([als, pad], axis=1)
    ad_ref[...] = jnp.concatenate([ald, pad], axis=1)
    mh = jnp.max(als, axis=0) + jnp.max(ald, axis=0)
    mh_ref[...] = jnp.concatenate([mh, jnp.zeros((15,), jnp.float32)])[None, :]


def _tc_mid(p, b1, w2, asrc2, adst2):
    return pl.pallas_call(
        _mid_body,
        out_shape=[
            jax.ShapeDtypeStruct((NPAD, 16), jnp.float32),
            jax.ShapeDtypeStruct((NPAD, 16), jnp.float32),
            jax.ShapeDtypeStruct((NPAD, 16), jnp.float32),
            jax.ShapeDtypeStruct((1, 16), jnp.float32),
        ],
    )(p, b1, w2, asrc2, adst2)


def _final_body(q_ref, b2_ref, o_ref):
    o_ref[...] = q_ref[0, :N, :C2] + q_ref[1, :N, :C2] + b2_ref[...]


def _tc_final(q, b2):
    return pl.pallas_call(
        _final_body,
        out_shape=jax.ShapeDtypeStruct((N, C2), jnp.float32),
    )(q, b2.reshape(1, C2))


def _sc_pass1(epad, npt):
    nch = npt // CHUNK
    mesh = plsc.VectorSubcoreMesh(core_axis_name="c", subcore_axis_name="s")

    @functools.partial(
        pl.kernel,
        out_type=[
            jax.ShapeDtypeStruct((epad, 16), jnp.float32),
            jax.ShapeDtypeStruct((2, NPAD, 16), jnp.float32),
        ],
        mesh=mesh,
        scratch_types=[
            pltpu.VMEM((CHUNK,), jnp.int32),
            pltpu.VMEM((CHUNK,), jnp.int32),
            pltpu.VMEM((CHUNK, 16), jnp.float32),
            pltpu.VMEM((CHUNK, 16), jnp.float32),
            pltpu.VMEM((CHUNK, 16), jnp.float32),
            pltpu.VMEM((1, 16), jnp.float32),
            pltpu.VMEM_SHARED((NPAD, 16), jnp.float32),
            pltpu.SemaphoreType.DMA,
            pltpu.SemaphoreType.DMA,
        ],
    )
    def kern(as_hbm, ad_hbm, mh_hbm, src_hbm, dst_hbm, z_hbm,
             ex_hbm, den_hbm,
             srcv, dstv, asr, adr, exb, mhv, dacc, sem1, sem2):
        cid = lax.axis_index("c")
        sid = lax.axis_index("s")
        wid = cid * 16 + sid
        pltpu.sync_copy(z_hbm.at[pl.ds(sid * RPT, RPT)],
                        dacc.at[pl.ds(sid * RPT, RPT)])
        pltpu.sync_copy(mh_hbm, mhv)
        plsc.subcore_barrier()
        mh = mhv[0, :]

        @pl.loop(0, nch)
        def _chunk(ci):
            base = wid * npt + ci * CHUNK
            pltpu.sync_copy(src_hbm.at[pl.ds(base, CHUNK)], srcv)
            pltpu.sync_copy(dst_hbm.at[pl.ds(base, CHUNK)], dstv)
            cp1 = pltpu.async_copy(as_hbm.at[srcv], asr, sem1)
            cp2 = pltpu.async_copy(ad_hbm.at[dstv], adr, sem2)
            cp1.wait()
            cp2.wait()

            @pl.loop(0, CHUNK)
            def _edge(e):
                t = asr[e, :] + adr[e, :]
                u = jnp.maximum(t, t * 0.2) - mh
                exb[e, :] = jnp.exp(u)

            pltpu.sync_copy(exb, ex_hbm.at[pl.ds(base, CHUNK)])
            pltpu.sync_copy(exb, dacc.at[dstv], add=True)

        plsc.subcore_barrier()
        pltpu.sync_copy(dacc.at[pl.ds(sid * RPT, RPT)],
                        den_hbm.at[cid, pl.ds(sid * RPT, RPT)])

    return kern


def _sc_pass2(epad, npt, hv):
    nch = npt // CHUNK
    fdim = 16 * hv
    mesh = plsc.VectorSubcoreMesh(core_axis_name="c", subcore_axis_name="s")

    @functools.partial(
        pl.kernel,
        out_type=jax.ShapeDtypeStruct((2, NPAD, fdim), jnp.float32),
        mesh=mesh,
        scratch_types=[
            pltpu.VMEM((CHUNK,), jnp.int32),
            pltpu.VMEM((CHUNK,), jnp.int32),
            pltpu.VMEM((CHUNK, fdim), jnp.float32),
            pltpu.VMEM((CHUNK, 16), jnp.float32),
            pltpu.VMEM((CHUNK, 16), jnp.float32),
            pltpu.VMEM((CHUNK, fdim), jnp.float32),
            pltpu.VMEM_SHARED((NPAD, fdim), jnp.float32),
            pltpu.SemaphoreType.DMA,
            pltpu.SemaphoreType.DMA,
        ],
    )
    def kern(h_hbm, ex_hbm, r_hbm, src_hbm, dst_hbm, z_hbm,
             out_hbm,
             srcv, dstv, hr, exr, rr, gb, acc, sem1, sem2):
        cid = lax.axis_index("c")
        sid = lax.axis_index("s")
        wid = cid * 16 + sid
        pltpu.sync_copy(z_hbm.at[pl.ds(sid * RPT, RPT)],
                        acc.at[pl.ds(sid * RPT, RPT)])
        plsc.subcore_barrier()

        @pl.loop(0, nch)
        def _chunk(ci):
            base = wid * npt + ci * CHUNK
            pltpu.sync_copy(src_hbm.at[pl.ds(base, CHUNK)], srcv)
            pltpu.sync_copy(dst_hbm.at[pl.ds(base, CHUNK)], dstv)
            cp1 = pltpu.async_copy(h_hbm.at[srcv], hr, sem1)
            cp2 = pltpu.async_copy(r_hbm.at[dstv], rr, sem2)
            pltpu.sync_copy(ex_hbm.at[pl.ds(base, CHUNK)], exr)
            cp1.wait()
            cp2.wait()

            @pl.loop(0, CHUNK)
            def _edge(e):
                al = exr[e, :] * rr[e, :]
                for v in range(hv):
                    sp = jnp.broadcast_to(al[v], (16,))
                    gb[e, pl.ds(16 * v, 16)] = hr[e, pl.ds(16 * v, 16)] * sp

            pltpu.sync_copy(gb, acc.at[dstv], add=True)

        plsc.subcore_barrier()
        pltpu.sync_copy(acc.at[pl.ds(sid * RPT, RPT)],
                        out_hbm.at[cid, pl.ds(sid * RPT, RPT)])

    return kern


def _block_attn(a):
    h = a.shape[0]
    eye = jnp.eye(h, dtype=a.dtype)
    return (a[:, :, None] * eye[:, None, :]).reshape(h * a.shape[1], h)


def kernel(x, edge_index, W1, a_src1, a_dst1, b1, W2, a_src2, a_dst2, b2):
    e = edge_index.shape[1]
    etot = e + N
    npt = _cdiv(etot, NTILES * CHUNK) * CHUNK
    epad = npt * NTILES

    xp = jnp.concatenate([x, jnp.zeros((NPAD - N, D_IN), jnp.float32)], axis=0)
    loops = jnp.arange(N, dtype=jnp.int32)
    padv = jnp.full((epad - etot,), N, jnp.int32)
    src = jnp.concatenate([edge_index[0], loops, padv])
    dst = jnp.concatenate([edge_index[1], loops, padv])

    asrc1 = _block_attn(a_src1)
    adst1 = _block_attn(a_dst1)
    asrc2 = _block_attn(a_src2)
    adst2 = _block_attn(a_dst2)

    z16 = jnp.zeros((NPAD, 16), jnp.float32)
    z128 = jnp.zeros((NPAD, 128), jnp.float32)

    pass1 = _sc_pass1(epad, npt)
    pass2_l1 = _sc_pass2(epad, npt, 8)
    pass2_l2 = _sc_pass2(epad, npt, 1)

    h1, as1, ad1, mh1 = _tc_prep(xp, W1, asrc1, adst1, H1 * C1)
    ex1, den1 = pass1(as1, ad1, mh1, src, dst, z16)
    r1 = _tc_recip(den1)
    p1 = pass2_l1(h1, ex1, r1, src, dst, z128)

    h2, as2, ad2, mh2 = _tc_mid(p1, b1.reshape(1, H1 * C1), W2, asrc2, adst2)
    ex2, den2 = pass1(as2, ad2, mh2, src, dst, z16)
    r2 = _tc_recip(den2)
    p2 = pass2_l2(h2, ex2, r2, src, dst, z16)

    return _tc_final(p2, b2)


# speedup vs baseline: 35.6679x; 35.6679x over previous
import functools

import jax
import jax.numpy as jnp
from jax import lax
from jax.experimental import pallas as pl
from jax.experimental.pallas import tpu as pltpu
from jax.experimental.pallas import tpu_sc as plsc

N = 10000
NPAD = 10112
D_IN = 128
H1, C1 = 8, 16
H2, C2 = 1, 2
NEG = -1e30
CHUNK = 128
NTILES = 32
RPT = NPAD // 16

_HIGH = lax.Precision.HIGHEST


def _cdiv(a, b):
    return (a + b - 1) // b


def _prep_body(x_ref, w_ref, asrc_ref, adst_ref, h_ref, as_ref, ad_ref, mh_ref):
    x = x_ref[...]
    h = jnp.dot(x, w_ref[...], precision=_HIGH)
    h_ref[...] = h
    als = jnp.dot(h, asrc_ref[...], precision=_HIGH)
    ald = jnp.dot(h, adst_ref[...], precision=_HIGH)
    nh = als.shape[1]
    pad = jnp.full((als.shape[0], 16 - nh), NEG, jnp.float32)
    as_ref[...] = jnp.concatenate([als, pad], axis=1)
    ad_ref[...] = jnp.concatenate([ald, pad], axis=1)
    mh = jnp.max(als, axis=0) + jnp.max(ald, axis=0)
    mh_ref[...] = jnp.concatenate([mh, jnp.zeros((16 - nh,), jnp.float32)])[None, :]


def _tc_prep(xp, w, asrc, adst, hdim):
    return pl.pallas_call(
        _prep_body,
        out_shape=[
            jax.ShapeDtypeStruct((NPAD, hdim), jnp.float32),
            jax.ShapeDtypeStruct((NPAD, 16), jnp.float32),
            jax.ShapeDtypeStruct((NPAD, 16), jnp.float32),
            jax.ShapeDtypeStruct((1, 16), jnp.float32),
        ],
    )(xp, w, asrc, adst)


def _recip_body(den_ref, r_ref):
    r_ref[...] = 1.0 / (den_ref[0] + den_ref[1] + 1e-16)


def _tc_recip(den):
    return pl.pallas_call(
        _recip_body,
        out_shape=jax.ShapeDtypeStruct((NPAD, 16), jnp.float32),
    )(den)


def _mid_body(p_ref, b1_ref, w2_ref, asrc_ref, adst_ref,
              h2_ref, as_ref, ad_ref, mh_ref):
    o = p_ref[0] + p_ref[1] + b1_ref[...]
    o = jnp.where(o > 0, o, jnp.exp(jnp.minimum(o, 0.0)) - 1.0)
    h2 = jnp.dot(o, w2_ref[...], precision=_HIGH)
    h2_ref[...] = jnp.concatenate(
        [h2, jnp.zeros((h2.shape[0], 16 - h2.shape[1]), jnp.float32)], axis=1)
    als = jnp.dot(h2, asrc_ref[...], precision=_HIGH)
    ald = jnp.dot(h2, adst_ref[...], precision=_HIGH)
    pad = jnp.full((als.shape[0], 15), NEG, jnp.float32)
    as_ref[...] = jnp.concatenate([als, pad], axis=1)
    ad_ref[...] = jnp.concatenate([ald, pad], axis=1)
    mh = jnp.max(als, axis=0) + jnp.max(ald, axis=0)
    mh_ref[...] = jnp.concatenate([mh, jnp.zeros((15,), jnp.float32)])[None, :]


def _tc_mid(p, b1, w2, asrc2, adst2):
    return pl.pallas_call(
        _mid_body,
        out_shape=[
            jax.ShapeDtypeStruct((NPAD, 16), jnp.float32),
            jax.ShapeDtypeStruct((NPAD, 16), jnp.float32),
            jax.ShapeDtypeStruct((NPAD, 16), jnp.float32),
            jax.ShapeDtypeStruct((1, 16), jnp.float32),
        ],
    )(p, b1, w2, asrc2, adst2)


def _final_body(q_ref, b2_ref, o_ref):
    o_ref[...] = q_ref[0, :N, :C2] + q_ref[1, :N, :C2] + b2_ref[...]


def _tc_final(q, b2):
    return pl.pallas_call(
        _final_body,
        out_shape=jax.ShapeDtypeStruct((N, C2), jnp.float32),
    )(q, b2.reshape(1, C2))


def _sc_pass1(epad, npt):
    nch = npt // CHUNK
    mesh = plsc.VectorSubcoreMesh(core_axis_name="c", subcore_axis_name="s")

    @functools.partial(
        pl.kernel,
        out_type=[
            jax.ShapeDtypeStruct((epad, 16), jnp.float32),
            jax.ShapeDtypeStruct((2, NPAD, 16), jnp.float32),
        ],
        mesh=mesh,
        compiler_params=pltpu.CompilerParams(use_tc_tiling_on_sc=False),
        scratch_types=[
            pltpu.VMEM((CHUNK,), jnp.int32),
            pltpu.VMEM((CHUNK,), jnp.int32),
            pltpu.VMEM((CHUNK, 16), jnp.float32),
            pltpu.VMEM((CHUNK, 16), jnp.float32),
            pltpu.VMEM((CHUNK, 16), jnp.float32),
            pltpu.VMEM((1, 16), jnp.float32),
            pltpu.VMEM_SHARED((NPAD, 16), jnp.float32),
            pltpu.SemaphoreType.DMA,
            pltpu.SemaphoreType.DMA,
        ],
    )
    def kern(as_hbm, ad_hbm, mh_hbm, src_hbm, dst_hbm, z_hbm,
             ex_hbm, den_hbm,
             srcv, dstv, asr, adr, exb, mhv, dacc, sem1, sem2):
        cid = lax.axis_index("c")
        sid = lax.axis_index("s")
        wid = cid * 16 + sid
        pltpu.sync_copy(z_hbm.at[pl.ds(sid * RPT, RPT)],
                        dacc.at[pl.ds(sid * RPT, RPT)])
        pltpu.sync_copy(mh_hbm, mhv)
        plsc.subcore_barrier()
        mh = mhv[0, :]

        @pl.loop(0, nch)
        def _chunk(ci):
            base = wid * npt + ci * CHUNK
            pltpu.sync_copy(src_hbm.at[pl.ds(base, CHUNK)], srcv)
            pltpu.sync_copy(dst_hbm.at[pl.ds(base, CHUNK)], dstv)
            cp1 = pltpu.async_copy(as_hbm.at[srcv], asr, sem1)
            cp2 = pltpu.async_copy(ad_hbm.at[dstv], adr, sem2)
            cp1.wait()
            cp2.wait()

            @pl.loop(0, CHUNK)
            def _edge(e):
                t = asr[e, :] + adr[e, :]
                u = jnp.maximum(t, t * 0.2) - mh
                exb[e, :] = jnp.exp(u)

            pltpu.sync_copy(exb, ex_hbm.at[pl.ds(base, CHUNK)])
            pltpu.sync_copy(exb, dacc.at[dstv], add=True)

        plsc.subcore_barrier()
        pltpu.sync_copy(dacc.at[pl.ds(sid * RPT, RPT)],
                        den_hbm.at[cid, pl.ds(sid * RPT, RPT)])

    return kern


def _sc_pass2(epad, npt, hv):
    nch = npt // CHUNK
    fdim = 16 * hv
    mesh = plsc.VectorSubcoreMesh(core_axis_name="c", subcore_axis_name="s")

    @functools.partial(
        pl.kernel,
        out_type=jax.ShapeDtypeStruct((2, NPAD, fdim), jnp.float32),
        mesh=mesh,
        compiler_params=pltpu.CompilerParams(use_tc_tiling_on_sc=False),
        scratch_types=[
            pltpu.VMEM((CHUNK,), jnp.int32),
            pltpu.VMEM((CHUNK,), jnp.int32),
            pltpu.VMEM((CHUNK, fdim), jnp.float32),
            pltpu.VMEM((CHUNK, 16), jnp.float32),
            pltpu.VMEM((CHUNK, 16), jnp.float32),
            pltpu.VMEM((CHUNK, fdim), jnp.float32),
            pltpu.VMEM_SHARED((NPAD, fdim), jnp.float32),
            pltpu.SemaphoreType.DMA,
            pltpu.SemaphoreType.DMA,
        ],
    )
    def kern(h_hbm, ex_hbm, r_hbm, src_hbm, dst_hbm, z_hbm,
             out_hbm,
             srcv, dstv, hr, exr, rr, gb, acc, sem1, sem2):
        cid = lax.axis_index("c")
        sid = lax.axis_index("s")
        wid = cid * 16 + sid
        pltpu.sync_copy(z_hbm.at[pl.ds(sid * RPT, RPT)],
                        acc.at[pl.ds(sid * RPT, RPT)])
        plsc.subcore_barrier()

        @pl.loop(0, nch)
        def _chunk(ci):
            base = wid * npt + ci * CHUNK
            pltpu.sync_copy(src_hbm.at[pl.ds(base, CHUNK)], srcv)
            pltpu.sync_copy(dst_hbm.at[pl.ds(base, CHUNK)], dstv)
            cp1 = pltpu.async_copy(h_hbm.at[srcv], hr, sem1)
            cp2 = pltpu.async_copy(r_hbm.at[dstv], rr, sem2)
            pltpu.sync_copy(ex_hbm.at[pl.ds(base, CHUNK)], exr)
            cp1.wait()
            cp2.wait()

            @pl.loop(0, CHUNK)
            def _edge(e):
                al = exr[e, :] * rr[e, :]
                for v in range(hv):
                    sp = jnp.broadcast_to(al[v], (16,))
                    gb[e, pl.ds(16 * v, 16)] = hr[e, pl.ds(16 * v, 16)] * sp

            pltpu.sync_copy(gb, acc.at[dstv], add=True)

        plsc.subcore_barrier()
        pltpu.sync_copy(acc.at[pl.ds(sid * RPT, RPT)],
                        out_hbm.at[cid, pl.ds(sid * RPT, RPT)])

    return kern


def _block_attn(a):
    h = a.shape[0]
    eye = jnp.eye(h, dtype=a.dtype)
    return (a[:, :, None] * eye[:, None, :]).reshape(h * a.shape[1], h)


def kernel(x, edge_index, W1, a_src1, a_dst1, b1, W2, a_src2, a_dst2, b2):
    e = edge_index.shape[1]
    etot = e + N
    npt = _cdiv(etot, NTILES * CHUNK) * CHUNK
    epad = npt * NTILES

    xp = jnp.concatenate([x, jnp.zeros((NPAD - N, D_IN), jnp.float32)], axis=0)
    loops = jnp.arange(N, dtype=jnp.int32)
    padv = jnp.full((epad - etot,), N, jnp.int32)
    src = jnp.concatenate([edge_index[0], loops, padv])
    dst = jnp.concatenate([edge_index[1], loops, padv])

    asrc1 = _block_attn(a_src1)
    adst1 = _block_attn(a_dst1)
    asrc2 = _block_attn(a_src2)
    adst2 = _block_attn(a_dst2)

    z16 = jnp.zeros((NPAD, 16), jnp.float32)
    z128 = jnp.zeros((NPAD, 128), jnp.float32)

    pass1 = _sc_pass1(epad, npt)
    pass2_l1 = _sc_pass2(epad, npt, 8)
    pass2_l2 = _sc_pass2(epad, npt, 1)

    h1, as1, ad1, mh1 = _tc_prep(xp, W1, asrc1, adst1, H1 * C1)
    ex1, den1 = pass1(as1, ad1, mh1, src, dst, z16)
    r1 = _tc_recip(den1)
    p1 = pass2_l1(h1, ex1, r1, src, dst, z128)

    h2, as2, ad2, mh2 = _tc_mid(p1, b1.reshape(1, H1 * C1), W2, asrc2, adst2)
    ex2, den2 = pass1(as2, ad2, mh2, src, dst, z16)
    r2 = _tc_recip(den2)
    p2 = pass2_l2(h2, ex2, r2, src, dst, z16)

    return _tc_final(p2, b2)


# speedup vs baseline: 50.3158x; 1.4107x over previous
import functools

import jax
import jax.numpy as jnp
from jax import lax
from jax.experimental import pallas as pl
from jax.experimental.pallas import tpu as pltpu
from jax.experimental.pallas import tpu_sc as plsc

N = 10000
NPAD = 10112
D_IN = 128
H1, C1 = 8, 16
H2, C2 = 1, 2
NEG = -1e30
CHUNK = 128
NTILES = 32
RPT = NPAD // 16

_HIGH = lax.Precision.HIGHEST


def _cdiv(a, b):
    return (a + b - 1) // b


def _prep_body(x_ref, w_ref, asrc_ref, adst_ref, pk_ref, ad_ref, mh_ref):
    x = x_ref[...]
    h = jnp.dot(x, w_ref[...], precision=_HIGH)
    als = jnp.dot(h, asrc_ref[...], precision=_HIGH)
    ald = jnp.dot(h, adst_ref[...], precision=_HIGH)
    pad = jnp.full((NPAD, 16 - H1), NEG, jnp.float32)
    pk_ref[...] = jnp.concatenate([als, pad, h], axis=1)
    ad_ref[...] = jnp.concatenate([ald, pad], axis=1)
    mh = jnp.max(als, axis=0) + jnp.max(ald, axis=0)
    mh_ref[...] = jnp.concatenate([mh, jnp.zeros((16 - H1,), jnp.float32)])[None, :]


def _tc_prep(xp, w, asrc, adst):
    return pl.pallas_call(
        _prep_body,
        out_shape=[
            jax.ShapeDtypeStruct((NPAD, 16 + H1 * C1), jnp.float32),
            jax.ShapeDtypeStruct((NPAD, 16), jnp.float32),
            jax.ShapeDtypeStruct((1, 16), jnp.float32),
        ],
    )(xp, w, asrc, adst)


BR = NPAD // 8


def _mid_body(q_ref, b1_ref, w2_ref, asrc_ref, adst_ref,
              pk_ref, ad_ref, mh_ref, ms_ref, md_ref):
    @pl.when(pl.program_id(0) == 0)
    def _init():
        ms_ref[...] = jnp.full((1, 16), NEG, jnp.float32)
        md_ref[...] = jnp.full((1, 16), NEG, jnp.float32)

    den = q_ref[0, :, :16] + q_ref[1, :, :16]
    feat = q_ref[0, :, 16:] + q_ref[1, :, 16:]
    r = 1.0 / (den[:, :H1] + 1e-16)
    rrep = jnp.reshape(
        jnp.broadcast_to(r[:, :, None], (BR, H1, C1)), (BR, H1 * C1))
    o = feat * rrep + b1_ref[...]
    o = jnp.where(o > 0, o, jnp.exp(jnp.minimum(o, 0.0)) - 1.0)
    h2 = jnp.dot(o, w2_ref[...], precision=_HIGH)
    als = jnp.dot(h2, asrc_ref[...], precision=_HIGH)
    ald = jnp.dot(h2, adst_ref[...], precision=_HIGH)
    negpad = jnp.full((BR, 15), NEG, jnp.float32)
    h2pad = jnp.concatenate(
        [h2, jnp.zeros((BR, 16 - C2), jnp.float32)], axis=1)
    pk_ref[...] = jnp.concatenate([als, negpad, h2pad], axis=1)
    ad_ref[...] = jnp.concatenate([ald, negpad], axis=1)
    ms_ref[...] = jnp.maximum(ms_ref[...], jnp.max(als))
    md_ref[...] = jnp.maximum(md_ref[...], jnp.max(ald))
    mh_ref[...] = ms_ref[...] + md_ref[...]


def _tc_mid(q, b1, w2, asrc2, adst2):
    return pl.pallas_call(
        _mid_body,
        grid=(NPAD // BR,),
        in_specs=[
            pl.BlockSpec((2, BR, 16 + H1 * C1), lambda i: (0, i, 0)),
            pl.BlockSpec((1, H1 * C1), lambda i: (0, 0)),
            pl.BlockSpec((H1 * C1, C2), lambda i: (0, 0)),
            pl.BlockSpec((C2, H2), lambda i: (0, 0)),
            pl.BlockSpec((C2, H2), lambda i: (0, 0)),
        ],
        out_specs=[
            pl.BlockSpec((BR, 32), lambda i: (i, 0)),
            pl.BlockSpec((BR, 16), lambda i: (i, 0)),
            pl.BlockSpec((1, 16), lambda i: (0, 0)),
        ],
        out_shape=[
            jax.ShapeDtypeStruct((NPAD, 32), jnp.float32),
            jax.ShapeDtypeStruct((NPAD, 16), jnp.float32),
            jax.ShapeDtypeStruct((1, 16), jnp.float32),
        ],
        scratch_shapes=[
            pltpu.VMEM((1, 16), jnp.float32),
            pltpu.VMEM((1, 16), jnp.float32),
        ],
    )(q, b1, w2, asrc2, adst2)


def _final_body(q_ref, b2_ref, o_ref):
    den = q_ref[0, :N, 0:1] + q_ref[1, :N, 0:1]
    feat = q_ref[0, :N, 16:16 + C2] + q_ref[1, :N, 16:16 + C2]
    o_ref[...] = feat / (den + 1e-16) + b2_ref[...]


def _tc_final(q, b2):
    return pl.pallas_call(
        _final_body,
        out_shape=jax.ShapeDtypeStruct((N, C2), jnp.float32),
    )(q, b2.reshape(1, C2))


def _sc_fused(epad, npt, hv):
    nch = npt // CHUNK
    aw = 16 * (hv + 1)
    mesh = plsc.VectorSubcoreMesh(core_axis_name="c", subcore_axis_name="s")

    @functools.partial(
        pl.kernel,
        out_type=jax.ShapeDtypeStruct((2, NPAD, aw), jnp.float32),
        mesh=mesh,
        compiler_params=pltpu.CompilerParams(use_tc_tiling_on_sc=False),
        scratch_types=[
            pltpu.VMEM((CHUNK,), jnp.int32),
            pltpu.VMEM((CHUNK,), jnp.int32),
            pltpu.VMEM((CHUNK, aw), jnp.float32),
            pltpu.VMEM((CHUNK, 16), jnp.float32),
            pltpu.VMEM((CHUNK, aw), jnp.float32),
            pltpu.VMEM((1, 16), jnp.float32),
            pltpu.VMEM_SHARED((NPAD, aw), jnp.float32),
            pltpu.SemaphoreType.DMA,
            pltpu.SemaphoreType.DMA,
        ],
    )
    def kern(pk_hbm, ad_hbm, mh_hbm, src_hbm, dst_hbm, z_hbm,
             out_hbm,
             srcv, dstv, pkr, adr, ob, mhv, acc, sem1, sem2):
        cid = lax.axis_index("c")
        sid = lax.axis_index("s")
        wid = cid * 16 + sid
        pltpu.sync_copy(z_hbm.at[pl.ds(sid * RPT, RPT)],
                        acc.at[pl.ds(sid * RPT, RPT)])
        pltpu.sync_copy(mh_hbm, mhv)
        plsc.subcore_barrier()
        mh = mhv[0, :]

        @pl.loop(0, nch)
        def _chunk(ci):
            base = wid * npt + ci * CHUNK
            pltpu.sync_copy(src_hbm.at[pl.ds(base, CHUNK)], srcv)
            pltpu.sync_copy(dst_hbm.at[pl.ds(base, CHUNK)], dstv)
            cp1 = pltpu.async_copy(pk_hbm.at[srcv], pkr, sem1)
            cp2 = pltpu.async_copy(ad_hbm.at[dstv], adr, sem2)
            cp1.wait()
            cp2.wait()

            @pl.loop(0, CHUNK)
            def _edge(e):
                t = pkr[e, pl.ds(0, 16)] + adr[e, :]
                u = jnp.maximum(t, t * 0.2) - mh
                ex = jnp.exp(u)
                ob[e, pl.ds(0, 16)] = ex
                for v in range(hv):
                    sp = jnp.broadcast_to(ex[v], (16,))
                    ob[e, pl.ds(16 * (v + 1), 16)] = (
                        pkr[e, pl.ds(16 * (v + 1), 16)] * sp)

            pltpu.sync_copy(ob, acc.at[dstv], add=True)

        plsc.subcore_barrier()
        pltpu.sync_copy(acc.at[pl.ds(sid * RPT, RPT)],
                        out_hbm.at[cid, pl.ds(sid * RPT, RPT)])

    return kern


def _block_attn(a):
    h = a.shape[0]
    eye = jnp.eye(h, dtype=a.dtype)
    return (a[:, :, None] * eye[:, None, :]).reshape(h * a.shape[1], h)


def kernel(x, edge_index, W1, a_src1, a_dst1, b1, W2, a_src2, a_dst2, b2):
    e = edge_index.shape[1]
    etot = e + N
    npt = _cdiv(etot, NTILES * CHUNK) * CHUNK
    epad = npt * NTILES

    xp = jnp.concatenate([x, jnp.zeros((NPAD - N, D_IN), jnp.float32)], axis=0)
    loops = jnp.arange(N, dtype=jnp.int32)
    padv = jnp.full((epad - etot,), N, jnp.int32)
    src = jnp.concatenate([edge_index[0], loops, padv])
    dst = jnp.concatenate([edge_index[1], loops, padv])

    asrc1 = _block_attn(a_src1)
    adst1 = _block_attn(a_dst1)
    asrc2 = _block_attn(a_src2)
    adst2 = _block_attn(a_dst2)

    z144 = jnp.zeros((NPAD, 16 + H1 * C1), jnp.float32)
    z32 = jnp.zeros((NPAD, 32), jnp.float32)

    fused1 = _sc_fused(epad, npt, H1)
    fused2 = _sc_fused(epad, npt, 1)

    pk1, ad1, mh1 = _tc_prep(xp, W1, asrc1, adst1)
    q1 = fused1(pk1, ad1, mh1, src, dst, z144)

    pk2, ad2, mh2 = _tc_mid(q1, b1.reshape(1, H1 * C1), W2, asrc2, adst2)
    q2 = fused2(pk2, ad2, mh2, src, dst, z32)

    return _tc_final(q2, b2)
